# baseline (speedup 1.0000x reference)
import functools

import jax
import jax.numpy as jnp
import numpy as np
from jax.experimental import pallas as pl
from jax.experimental.pallas import tpu as pltpu

_C1 = 0.01 ** 2
_C2 = 0.03 ** 2


def _reflect_matrix(n):
    m = np.zeros((n, n), dtype=np.float32)
    for j in range(n):
        for i in (j - 1, j, j + 1):
            if 0 <= i < n:
                m[j, i] += 1.0
    m[0, 1] += 1.0
    m[n - 1, n - 2] += 1.0
    return m


def _split_hi_lo(a):
    hi = a.astype(jnp.bfloat16)
    lo = (a - hi.astype(jnp.float32)).astype(jnp.bfloat16)
    return hi, lo


def _ssim_body(x_ref, y_ref, m_ref, mt_ref, o_ref, *, bt, img_h, img_w):
    mrow = m_ref[...]
    mcol = mt_ref[...]
    x = x_ref[...].reshape(bt * img_h, img_w)
    y = y_ref[...].reshape(bt * img_h, img_w)

    pp = x * x + y * y
    qq = x * y

    def colpass(a):
        hi, lo = _split_hi_lo(a)
        return (jnp.dot(hi, mcol, preferred_element_type=jnp.float32)
                + jnp.dot(lo, mcol, preferred_element_type=jnp.float32))

    cx = colpass(x)
    cy = colpass(y)
    cp = colpass(pp)
    cq = colpass(qq)

    k1 = jnp.float32(81.0 * _C1)
    k2 = jnp.float32(81.0 * _C2)
    for b in range(bt):
        sl = slice(b * img_h, (b + 1) * img_h)

        def rowpass(c):
            hi, lo = _split_hi_lo(c[sl])
            return (jnp.dot(mrow, hi, preferred_element_type=jnp.float32)
                    + jnp.dot(mrow, lo, preferred_element_type=jnp.float32))

        sx = rowpass(cx)
        sy = rowpass(cy)
        sp = rowpass(cp)
        sq = rowpass(cq)

        a2 = sx * sy
        b2 = sx * sx + sy * sy
        n1 = 2.0 * a2 + k1
        n2 = 18.0 * sq - 2.0 * a2 + k2
        d1 = b2 + k1
        d2 = 9.0 * sp - b2 + k2
        out = jnp.clip(
            0.5 - 0.5 * (n1 * n2) * pl.reciprocal(d1 * d2, approx=True),
            0.0, 1.0)
        o_ref[b] = out.astype(o_ref.dtype)


def _pick_block_images(batch, cap_images):
    divs = [d for d in range(1, batch + 1)
            if batch % d == 0 and d <= cap_images]
    return max(divs) if divs else 1


def kernel(x, y):
    assert x.shape == y.shape and x.ndim == 4
    N, C, H, W = x.shape
    B = N * C

    bt = _pick_block_images(B, 12)
    g = B // bt

    mrow = jnp.asarray(_reflect_matrix(H), dtype=jnp.bfloat16)
    mcol = jnp.asarray(_reflect_matrix(W).T, dtype=jnp.bfloat16)

    xb = x.reshape(B, H, W)
    yb = y.reshape(B, H, W)
    spec = pl.BlockSpec((bt, H, W), lambda i: (i, 0, 0))
    mspec = pl.BlockSpec((H, H), lambda i: (0, 0))
    mtspec = pl.BlockSpec((W, W), lambda i: (0, 0))
    out = pl.pallas_call(
        functools.partial(_ssim_body, bt=bt, img_h=H, img_w=W),
        out_shape=jax.ShapeDtypeStruct((B, H, W), x.dtype),
        grid=(g,),
        in_specs=[spec, spec, mspec, mtspec],
        out_specs=spec,
        compiler_params=pltpu.CompilerParams(
            dimension_semantics=("parallel",),
            vmem_limit_bytes=56 * 1024 * 1024,
        ),
    )(xb, yb, mrow, mcol)
    return out.reshape(N, C, H, W)


# speedup vs baseline: 1.0229x; 1.0229x over previous
import functools

import jax
import jax.numpy as jnp
import numpy as np
from jax.experimental import pallas as pl
from jax.experimental.pallas import tpu as pltpu

_C1 = 0.01 ** 2
_C2 = 0.03 ** 2


def _reflect_matrix(n):
    m = np.zeros((n, n), dtype=np.float32)
    for j in range(n):
        for i in (j - 1, j, j + 1):
            if 0 <= i < n:
                m[j, i] += 1.0
    m[0, 1] += 1.0
    m[n - 1, n - 2] += 1.0
    return m


def _split_hi_lo(a):
    hi = a.astype(jnp.bfloat16)
    lo = (a - hi.astype(jnp.float32)).astype(jnp.bfloat16)
    return hi, lo


def _ssim_body(x_ref, y_ref, m_ref, mt_ref, o_ref, chi_ref, clo_ref,
               *, bt, img_h, img_w):
    mrow = m_ref[...]
    mcol = mt_ref[...]
    x = x_ref[...].reshape(bt * img_h, img_w)
    y = y_ref[...].reshape(bt * img_h, img_w)

    pp = x * x + y * y
    qq = x * y

    for qi, a in enumerate((x, y, pp, qq)):
        hi, lo = _split_hi_lo(a)
        c = (jnp.dot(hi, mcol, preferred_element_type=jnp.float32)
             + jnp.dot(lo, mcol, preferred_element_type=jnp.float32))
        chi, clo = _split_hi_lo(c)
        chi_ref[:, qi * img_w:(qi + 1) * img_w] = chi
        clo_ref[:, qi * img_w:(qi + 1) * img_w] = clo

    k1 = jnp.float32(81.0 * _C1)
    k2 = jnp.float32(81.0 * _C2)
    for b in range(bt):
        sl = pl.ds(b * img_h, img_h)
        r = (jnp.dot(mrow, chi_ref[sl, :], preferred_element_type=jnp.float32)
             + jnp.dot(mrow, clo_ref[sl, :], preferred_element_type=jnp.float32))
        sx = r[:, 0:img_w]
        sy = r[:, img_w:2 * img_w]
        sp = r[:, 2 * img_w:3 * img_w]
        sq = r[:, 3 * img_w:4 * img_w]

        a2 = sx * sy
        b2 = sx * sx + sy * sy
        n1 = 2.0 * a2 + k1
        n2 = 18.0 * sq - 2.0 * a2 + k2
        d1 = b2 + k1
        d2 = 9.0 * sp - b2 + k2
        out = jnp.clip(
            0.5 - 0.5 * (n1 * n2) * pl.reciprocal(d1 * d2, approx=True),
            0.0, 1.0)
        o_ref[b] = out.astype(o_ref.dtype)


def _pick_block_images(batch, cap_images):
    divs = [d for d in range(1, batch + 1)
            if batch % d == 0 and d <= cap_images]
    return max(divs) if divs else 1


def kernel(x, y):
    assert x.shape == y.shape and x.ndim == 4
    N, C, H, W = x.shape
    B = N * C

    bt = _pick_block_images(B, 8)
    g = B // bt

    mrow = jnp.asarray(_reflect_matrix(H), dtype=jnp.bfloat16)
    mcol = jnp.asarray(_reflect_matrix(W).T, dtype=jnp.bfloat16)

    xb = x.reshape(B, H, W)
    yb = y.reshape(B, H, W)
    spec = pl.BlockSpec((bt, H, W), lambda i: (i, 0, 0))
    mspec = pl.BlockSpec((H, H), lambda i: (0, 0))
    mtspec = pl.BlockSpec((W, W), lambda i: (0, 0))
    out = pl.pallas_call(
        functools.partial(_ssim_body, bt=bt, img_h=H, img_w=W),
        out_shape=jax.ShapeDtypeStruct((B, H, W), x.dtype),
        grid=(g,),
        in_specs=[spec, spec, mspec, mtspec],
        out_specs=spec,
        scratch_shapes=[
            pltpu.VMEM((bt * H, 4 * W), jnp.bfloat16),
            pltpu.VMEM((bt * H, 4 * W), jnp.bfloat16),
        ],
        compiler_params=pltpu.CompilerParams(
            dimension_semantics=("parallel",),
            vmem_limit_bytes=56 * 1024 * 1024,
        ),
    )(xb, yb, mrow, mcol)
    return out.reshape(N, C, H, W)
